# baseline (speedup 1.0000x reference)
import functools

import jax
import jax.numpy as jnp
from jax import lax
from jax.experimental import pallas as pl
from jax.experimental.pallas import tpu as pltpu
from jax.experimental.pallas import tpu_sc as plsc

DIM = 64
NUM_CORES = 2
NUM_SUBCORES = 16
NW = NUM_CORES * NUM_SUBCORES
LANES = 16
KD = DIM // LANES

_INV_PIO2 = 0.6366197723675814
_PIO2_HI = 1.5707963705062866
_PIO2_LO = -4.371139000186241e-08
_S3 = -1.6666654611e-01
_S5 = 8.3321608736e-03
_C2 = -4.9999997019e-01
_C4 = 4.1659855842e-02
_C6 = -1.3585052083e-03


def _sincos16(a):
    z = a * _INV_PIO2
    q = (z + 0.5 * jnp.sign(z)).astype(jnp.int32)
    n = q.astype(jnp.float32)
    x = (a - n * _PIO2_HI) - n * _PIO2_LO
    x2 = x * x
    sp = x * (1.0 + x2 * (_S3 + x2 * _S5))
    cp = 1.0 + x2 * (_C2 + x2 * (_C4 + x2 * _C6))
    b0 = (q & 1) != 0
    b1 = (q & 2) != 0
    bx = ((q ^ (q >> 1)) & 1) != 0
    s_r = jnp.where(b0, cp, sp)
    c_r = jnp.where(b0, sp, cp)
    s = jnp.where(b1, -s_r, s_r)
    c = jnp.where(bx, -c_r, c_r)
    return s, c


def _sc_rotate(emb_real, emb_imag, idx3, wa, h, t, ta, B, L):
    bpw = B // NW
    mesh = plsc.VectorSubcoreMesh(core_axis_name="c", subcore_axis_name="s")

    @functools.partial(
        pl.kernel,
        mesh=mesh,
        out_type=(jax.ShapeDtypeStruct((B, L, DIM), jnp.float32),
                  jax.ShapeDtypeStruct((B, L, DIM), jnp.float32)),
        scratch_types=(
            pltpu.VMEM((bpw, L), jnp.int32),
            pltpu.VMEM((2, L, DIM), jnp.float32),
            pltpu.VMEM((2, L, DIM), jnp.float32),
            pltpu.VMEM((2, L, DIM), jnp.float32),
            pltpu.VMEM((2, L, DIM), jnp.float32),
            pltpu.VMEM((2, L, LANES), jnp.float32),
            pltpu.VMEM((DIM,), jnp.float32),
            pltpu.VMEM((2, L, DIM), jnp.float32),
            pltpu.VMEM((2, L, DIM), jnp.float32),
            pltpu.SemaphoreType.DMA,
            pltpu.SemaphoreType.DMA,
            pltpu.SemaphoreType.DMA,
            pltpu.SemaphoreType.DMA,
        ),
        compiler_params=pltpu.CompilerParams(use_tc_tiling_on_sc=False),
    )
    def rot_kernel(er_hbm, ei_hbm, idx_hbm, wa_hbm, h_hbm, t_hbm, ta_hbm,
                   or_hbm, oi_hbm, idx_v, er_v, ei_v, wa_v, h_v, t_v, ta_v,
                   or_v, oi_v, sem_in0, sem_in1, sem_out0, sem_out1):
        wid = lax.axis_index("s") * NUM_CORES + lax.axis_index("c")
        b0w = wid * bpw
        pltpu.sync_copy(idx_hbm.at[wid], idx_v)
        pltpu.sync_copy(ta_hbm, ta_v)

        def in_descriptors(g, slot, sem):
            b = b0w + g
            return (
                pltpu.make_async_copy(er_hbm.at[idx_v.at[g]], er_v.at[slot], sem),
                pltpu.make_async_copy(ei_hbm.at[idx_v.at[g]], ei_v.at[slot], sem),
                pltpu.make_async_copy(wa_hbm.at[b], wa_v.at[slot], sem),
                pltpu.make_async_copy(h_hbm.at[b], h_v.at[slot], sem),
                pltpu.make_async_copy(t_hbm.at[b], t_v.at[slot], sem),
            )

        def prefetch(g, slot, sem):
            for cp in in_descriptors(g, slot, sem):
                cp.start()

        def out_descriptors(g, slot, sem):
            return (
                pltpu.make_async_copy(or_v.at[slot], or_hbm.at[b0w + g], sem),
                pltpu.make_async_copy(oi_v.at[slot], oi_hbm.at[b0w + g], sem),
            )

        def compute_store(g, slot, sem_in, sem_out, drain_out):
            for cp in in_descriptors(g, slot, sem_in):
                cp.wait()

            def lbody(l, carry):
                tvec = t_v[slot, l, pl.ds(0, LANES)]
                for k in range(KD):
                    sl = pl.ds(k * LANES, LANES)
                    av = (tvec * (ta_v[sl] + wa_v[slot, l, sl])
                          + h_v[slot, l, sl])
                    s, c = _sincos16(av)
                    er = er_v[slot, l, sl]
                    ei = ei_v[slot, l, sl]
                    or_v[slot, l, sl] = er * c - ei * s
                    oi_v[slot, l, sl] = er * s + ei * c
                return carry

            @pl.when(drain_out)
            def _():
                for cp in out_descriptors(g, slot, sem_out):
                    cp.wait()

            lax.fori_loop(0, L, lbody, 0)
            for cp in out_descriptors(g, slot, sem_out):
                cp.start()

        prefetch(0, 0, sem_in0)

        def body(p, carry):
            g0 = 2 * p
            prefetch(g0 + 1, 1, sem_in1)
            compute_store(g0, 0, sem_in0, sem_out0, p > 0)

            @pl.when(g0 + 2 < bpw)
            def _():
                prefetch(g0 + 2, 0, sem_in0)

            compute_store(g0 + 1, 1, sem_in1, sem_out1, p > 0)
            return carry

        lax.fori_loop(0, bpw // 2, body, 0)
        for cp in out_descriptors(bpw - 2, 0, sem_out0):
            cp.wait()
        for cp in out_descriptors(bpw - 1, 1, sem_out1):
            cp.wait()

    return rot_kernel(emb_real, emb_imag, idx3, wa, h, t, ta)


def kernel(histories, sources, t, word_angles, emb_real, emb_imag,
           dimension_nums, rotary_denom):
    B, L, dim = histories.shape
    time_angle = (1.0 / rotary_denom ** (dimension_nums / dim)).astype(
        jnp.float32)

    idx3 = sources.reshape(NW, B // NW, L)
    t_rep = jnp.broadcast_to(t, (B, L, LANES))
    o_re, o_im = _sc_rotate(emb_real, emb_imag, idx3, word_angles, histories,
                            t_rep, time_angle, B, L)
    return lax.complex(o_re, o_im)


# speedup vs baseline: 1.2603x; 1.2603x over previous
import functools

import jax
import jax.numpy as jnp
from jax import lax
from jax.experimental import pallas as pl
from jax.experimental.pallas import tpu as pltpu
from jax.experimental.pallas import tpu_sc as plsc

DIM = 64
NUM_CORES = 2
NUM_SUBCORES = 16
NW = NUM_CORES * NUM_SUBCORES


def _sc_gather(emb_real, emb_imag, idx3, B, L):
    bpw = B // NW
    mesh = plsc.VectorSubcoreMesh(core_axis_name="c", subcore_axis_name="s")

    @functools.partial(
        pl.kernel,
        mesh=mesh,
        out_type=jax.ShapeDtypeStruct((B, L, 2 * DIM), jnp.float32),
        scratch_types=(
            pltpu.VMEM((bpw, L), jnp.int32),
            pltpu.VMEM((2, L, DIM), jnp.float32),
            pltpu.VMEM((2, L, DIM), jnp.float32),
            pltpu.SemaphoreType.DMA,
            pltpu.SemaphoreType.DMA,
            pltpu.SemaphoreType.DMA,
            pltpu.SemaphoreType.DMA,
        ),
        compiler_params=pltpu.CompilerParams(use_tc_tiling_on_sc=False),
    )
    def gather_kernel(er_hbm, ei_hbm, idx_hbm, ec_out,
                      idx_v, er_v, ei_v, sem_r0, sem_r1, sem_i0, sem_i1):
        wid = lax.axis_index("s") * NUM_CORES + lax.axis_index("c")
        b0 = wid * bpw
        pltpu.sync_copy(idx_hbm.at[wid], idx_v)

        def descriptors(g, slot, sem_r, sem_i):
            cp_r = pltpu.make_async_copy(
                er_hbm.at[idx_v.at[g]], er_v.at[slot], sem_r)
            cp_i = pltpu.make_async_copy(
                ei_hbm.at[idx_v.at[g]], ei_v.at[slot], sem_i)
            return cp_r, cp_i

        def gathers(g, slot, sem_r, sem_i):
            cp_r, cp_i = descriptors(g, slot, sem_r, sem_i)
            cp_r.start()
            cp_i.start()

        def stores(g, slot, sem_r, sem_i):
            cp_r, cp_i = descriptors(g, slot, sem_r, sem_i)
            cp_r.wait()
            cp_i.wait()
            pltpu.sync_copy(er_v.at[slot], ec_out.at[b0 + g, :, pl.ds(0, DIM)])
            pltpu.sync_copy(ei_v.at[slot], ec_out.at[b0 + g, :, pl.ds(DIM, DIM)])

        gathers(0, 0, sem_r0, sem_i0)

        def body(p, carry):
            g0 = 2 * p
            gathers(g0 + 1, 1, sem_r1, sem_i1)
            stores(g0, 0, sem_r0, sem_i0)

            @pl.when(g0 + 2 < bpw)
            def _():
                gathers(g0 + 2, 0, sem_r0, sem_i0)

            stores(g0 + 1, 1, sem_r1, sem_i1)
            return carry

        lax.fori_loop(0, bpw // 2, body, 0)

    return gather_kernel(emb_real, emb_imag, idx3)


_INV_PIO2 = 0.6366197723675814
_PIO2_HI = 1.5707963705062866
_PIO2_LO = -4.371139000186241e-08
_S3 = -1.6666654611e-01
_S5 = 8.3321608736e-03
_C2 = -4.9999997019e-01
_C4 = 4.1659855842e-02
_C6 = -1.3585052083e-03


def _sincos(a):
    n = jnp.round(a * _INV_PIO2)
    x = (a - n * _PIO2_HI) - n * _PIO2_LO
    x2 = x * x
    sp = x * (1.0 + x2 * (_S3 + x2 * _S5))
    cp = 1.0 + x2 * (_C2 + x2 * (_C4 + x2 * _C6))
    q = n.astype(jnp.int32)
    b0 = (q & 1) != 0
    b1 = (q & 2) != 0
    s_r = jnp.where(b0, cp, sp)
    c_r = jnp.where(b0, sp, cp)
    s = jnp.where(b1, -s_r, s_r)
    c = jnp.where(b0 != b1, -c_r, c_r)
    return s, c


def _rot_body(t_ref, ta_ref, wa_ref, h_ref, ec_ref, or_ref, oi_ref):
    a = t_ref[...] * (ta_ref[...] + wa_ref[...]) + h_ref[...]
    s, c = _sincos(a)
    er = ec_ref[..., :DIM]
    ei = ec_ref[..., DIM:]
    or_ref[...] = er * c - ei * s
    oi_ref[...] = er * s + ei * c


def _tc_rotate(t, ta3, wa, h, ec, B, L):
    bb = 64
    grid = (B // bb,)
    blk = lambda i: (i, 0, 0)
    zero = lambda i: (0, 0, 0)
    return pl.pallas_call(
        _rot_body,
        grid=grid,
        in_specs=[
            pl.BlockSpec((bb, L, 1), blk),
            pl.BlockSpec((1, 1, DIM), zero),
            pl.BlockSpec((bb, L, DIM), blk),
            pl.BlockSpec((bb, L, DIM), blk),
            pl.BlockSpec((bb, L, 2 * DIM), blk),
        ],
        out_specs=[pl.BlockSpec((bb, L, DIM), blk)] * 2,
        out_shape=[jax.ShapeDtypeStruct((B, L, DIM), jnp.float32)] * 2,
    )(t, ta3, wa, h, ec)


def kernel(histories, sources, t, word_angles, emb_real, emb_imag,
           dimension_nums, rotary_denom):
    B, L, dim = histories.shape
    time_angle = 1.0 / rotary_denom ** (dimension_nums / dim)

    idx3 = sources.reshape(NW, B // NW, L)
    ec = _sc_gather(emb_real, emb_imag, idx3, B, L)

    o_re, o_im = _tc_rotate(t, time_angle.reshape(1, 1, dim), word_angles,
                            histories, ec, B, L)
    return lax.complex(o_re, o_im)


# speedup vs baseline: 1.9016x; 1.5088x over previous
import functools

import jax
import jax.numpy as jnp
from jax import lax
from jax.experimental import pallas as pl
from jax.experimental.pallas import tpu as pltpu
from jax.experimental.pallas import tpu_sc as plsc

DIM = 64
NUM_CORES = 2
NUM_SUBCORES = 16
NW = NUM_CORES * NUM_SUBCORES


def _sc_gather(emb_real, emb_imag, idx3, B, L):
    bpw = B // NW
    mesh = plsc.VectorSubcoreMesh(core_axis_name="c", subcore_axis_name="s")

    @functools.partial(
        pl.kernel,
        mesh=mesh,
        out_type=jax.ShapeDtypeStruct((B, L, 2 * DIM), jnp.float32),
        scratch_types=(
            pltpu.VMEM((bpw, L), jnp.int32),
            pltpu.VMEM((2, L, DIM), jnp.float32),
            pltpu.VMEM((2, L, DIM), jnp.float32),
            pltpu.SemaphoreType.DMA,
            pltpu.SemaphoreType.DMA,
            pltpu.SemaphoreType.DMA,
            pltpu.SemaphoreType.DMA,
        ),
        compiler_params=pltpu.CompilerParams(use_tc_tiling_on_sc=False),
    )
    def gather_kernel(er_hbm, ei_hbm, idx_hbm, ec_out,
                      idx_v, er_v, ei_v, sem_r0, sem_r1, sem_i0, sem_i1):
        wid = lax.axis_index("s") * NUM_CORES + lax.axis_index("c")
        b0 = wid * bpw
        pltpu.sync_copy(idx_hbm.at[wid], idx_v)

        def descriptors(g, slot, sem_r, sem_i):
            cp_r = pltpu.make_async_copy(
                er_hbm.at[idx_v.at[g]], er_v.at[slot], sem_r)
            cp_i = pltpu.make_async_copy(
                ei_hbm.at[idx_v.at[g]], ei_v.at[slot], sem_i)
            return cp_r, cp_i

        def gathers(g, slot, sem_r, sem_i):
            cp_r, cp_i = descriptors(g, slot, sem_r, sem_i)
            cp_r.start()
            cp_i.start()

        def stores(g, slot, sem_r, sem_i):
            cp_r, cp_i = descriptors(g, slot, sem_r, sem_i)
            cp_r.wait()
            cp_i.wait()
            pltpu.sync_copy(er_v.at[slot], ec_out.at[b0 + g, :, pl.ds(0, DIM)])
            pltpu.sync_copy(ei_v.at[slot], ec_out.at[b0 + g, :, pl.ds(DIM, DIM)])

        gathers(0, 0, sem_r0, sem_i0)

        def body(p, carry):
            g0 = 2 * p
            gathers(g0 + 1, 1, sem_r1, sem_i1)
            stores(g0, 0, sem_r0, sem_i0)

            @pl.when(g0 + 2 < bpw)
            def _():
                gathers(g0 + 2, 0, sem_r0, sem_i0)

            stores(g0 + 1, 1, sem_r1, sem_i1)
            return carry

        lax.fori_loop(0, bpw // 2, body, 0)

    return gather_kernel(emb_real, emb_imag, idx3)


_INV_PIO2 = 0.6366197723675814
_PIO2_HI = 1.5707963705062866
_PIO2_LO = -4.371139000186241e-08
_S3 = -1.6666654611e-01
_S5 = 8.3321608736e-03
_C2 = -4.9999997019e-01
_C4 = 4.1659855842e-02
_C6 = -1.3585052083e-03


def _sincos(a):
    n = jnp.round(a * _INV_PIO2)
    x = (a - n * _PIO2_HI) - n * _PIO2_LO
    x2 = x * x
    sp = x * (1.0 + x2 * (_S3 + x2 * _S5))
    cp = 1.0 + x2 * (_C2 + x2 * (_C4 + x2 * _C6))
    q = n.astype(jnp.int32)
    b0 = (q & 1) != 0
    b1 = (q & 2) != 0
    s_r = jnp.where(b0, cp, sp)
    c_r = jnp.where(b0, sp, cp)
    s = jnp.where(b1, -s_r, s_r)
    c = jnp.where(b0 != b1, -c_r, c_r)
    return s, c


def _rot_body(t_ref, ta_ref, wa_ref, h_ref, ec_ref, or_ref, oi_ref):
    a = t_ref[...] * (ta_ref[...] + wa_ref[...]) + h_ref[...]
    s, c = _sincos(a)
    er = ec_ref[..., :DIM]
    ei = ec_ref[..., DIM:]
    or_ref[...] = er * c - ei * s
    oi_ref[...] = er * s + ei * c


def _tc_rotate(t, ta3, wa, h, ec, B, L, half, nhalves):
    bb = 64
    b2 = B // nhalves
    grid = (b2 // bb,)
    off = half * (b2 // bb)
    blk = lambda i: (i + off, 0, 0)
    loc = lambda i: (i, 0, 0)
    zero = lambda i: (0, 0, 0)
    return pl.pallas_call(
        _rot_body,
        grid=grid,
        in_specs=[
            pl.BlockSpec((bb, L, 1), blk),
            pl.BlockSpec((1, 1, DIM), zero),
            pl.BlockSpec((bb, L, DIM), blk),
            pl.BlockSpec((bb, L, DIM), blk),
            pl.BlockSpec((bb, L, 2 * DIM), loc),
        ],
        out_specs=[pl.BlockSpec((bb, L, DIM), loc)] * 2,
        out_shape=[jax.ShapeDtypeStruct((b2, L, DIM), jnp.float32)] * 2,
    )(t, ta3, wa, h, ec)


def kernel(histories, sources, t, word_angles, emb_real, emb_imag,
           dimension_nums, rotary_denom):
    B, L, dim = histories.shape
    time_angle = 1.0 / rotary_denom ** (dimension_nums / dim)

    nh = 2
    b2 = B // nh
    ta3 = time_angle.reshape(1, 1, dim)
    res, ims = [], []
    ecs = []
    for half in range(nh):
        idx3 = lax.slice_in_dim(sources, half * b2, (half + 1) * b2, axis=0)
        ecs.append(_sc_gather(emb_real, emb_imag,
                              idx3.reshape(NW, b2 // NW, L), b2, L))
    for half in range(nh):
        o_re, o_im = _tc_rotate(t, ta3, word_angles, histories, ecs[half],
                                B, L, half, nh)
        res.append(o_re)
        ims.append(o_im)
    return lax.complex(jnp.concatenate(res, axis=0),
                       jnp.concatenate(ims, axis=0))
